# baseline (speedup 1.0000x reference)
import functools
import math

import jax
import jax.numpy as jnp
from jax import lax
from jax.experimental import pallas as pl
from jax.experimental.pallas import tpu as pltpu
from jax.experimental.pallas import tpu_sc as plsc

_BASE_GSD = 10.0
_LN10K = math.log(10000.0)
_MONTH_SCALE = 2.0 * math.pi / 12.0
_LANES = 16


def _sc_gather_kernel(is_hbm, it_hbm, im_hbm, ce_hbm, pos_hbm, mtab_hbm,
                      a0_hbm, a1_hbm, a2_hbm,
                      is_v, it_v, im_v,
                      rows0_v, rows1_v, rows2_v,
                      sem0, sem1, sem2, *, num_cores, n_rows):
    wid = lax.axis_index("s") * num_cores + lax.axis_index("c")
    n_workers = n_rows // _LANES

    @pl.when(wid < n_workers)
    def _():
        base = wid * _LANES
        pltpu.sync_copy(is_hbm.at[pl.ds(base, _LANES)], is_v)
        pltpu.sync_copy(it_hbm.at[pl.ds(base, _LANES)], it_v)
        pltpu.sync_copy(im_hbm.at[pl.ds(base, _LANES)], im_v)
        cp0 = pltpu.async_copy(ce_hbm.at[is_v], rows0_v, sem0)
        cp1 = pltpu.async_copy(pos_hbm.at[it_v], rows1_v, sem1)
        cp2 = pltpu.async_copy(mtab_hbm.at[im_v], rows2_v, sem2)
        cp0.wait()
        cp1.wait()
        cp2.wait()
        pltpu.sync_copy(rows0_v, a0_hbm.at[pl.ds(base, _LANES)])
        pltpu.sync_copy(rows1_v, a1_hbm.at[pl.ds(base, _LANES)])
        pltpu.sync_copy(rows2_v, a2_hbm.at[pl.ds(base, _LANES)])


def _spatial_kernel(gsd_ref, s_ref, *, w_cnt):
    hw, n = s_ref.shape
    quarter = n // 4
    g = jax.lax.broadcasted_iota(jnp.int32, (hw, quarter), 0)
    lane_q = jax.lax.broadcasted_iota(
        jnp.int32, (hw, quarter), 1).astype(jnp.float32)
    gsd = gsd_ref[0, 0]
    omega_q = jnp.exp(lane_q * (-_LN10K / quarter))
    ph = (g // w_cnt).astype(jnp.float32) * gsd
    pw = (g % w_cnt).astype(jnp.float32) * gsd
    argh = ph * omega_q
    argw = pw * omega_q
    s_ref[...] = jnp.concatenate(
        [jnp.sin(argh), jnp.cos(argh), jnp.sin(argw), jnp.cos(argw)], axis=1)


def _stream_kernel(tok_ref, a_ref, s_ref, out_ref):
    n3 = a_ref.shape[-1]
    out_ref[..., :n3] = tok_ref[..., :n3] + a_ref[...][:, None]
    out_ref[..., n3:] = (tok_ref[..., n3:]
                         + s_ref[...][:, :, None, None, :][None])


def _sincos_1d_table(t_cnt, n):
    lane = jnp.arange(n // 2, dtype=jnp.float32)
    omega = jnp.exp(lane * (-_LN10K / (n // 2)))
    arg = jnp.arange(t_cnt, dtype=jnp.float32)[:, None] * omega[None, :]
    return jnp.concatenate([jnp.sin(arg), jnp.cos(arg)], axis=1)


def _month_table(n):
    ang = jnp.arange(12, dtype=jnp.float32) * _MONTH_SCALE
    return jnp.concatenate([
        jnp.broadcast_to(jnp.sin(ang)[:, None], (12, n // 2)),
        jnp.broadcast_to(jnp.cos(ang)[:, None], (12, n // 2))], axis=1)


def kernel(tokens, channel_embed, timestamps, patch_size, input_res):
    b, h, w, t, s, d = tokens.shape
    n = d // 4
    hw = h * w
    n_rows = b * t * s

    months = timestamps[:, :, 1].astype(jnp.int32).reshape(-1)
    r = jnp.arange(n_rows, dtype=jnp.int32)
    is_tab = r % s
    it_tab = (r // s) % t
    im_tab = jnp.repeat(months, s)
    gsd = (jnp.asarray(input_res, jnp.float32)
           * jnp.asarray(patch_size, jnp.float32) / _BASE_GSD).reshape(1, 1)

    pos_tab = _sincos_1d_table(t, n)
    mtab = _month_table(n)

    info = plsc.get_sparse_core_info()
    sc_gather = pl.kernel(
        functools.partial(
            _sc_gather_kernel, num_cores=info.num_cores, n_rows=n_rows),
        out_type=[jax.ShapeDtypeStruct((n_rows, n), jnp.float32)] * 3,
        mesh=plsc.VectorSubcoreMesh(core_axis_name="c", subcore_axis_name="s"),
        scratch_types=[
            pltpu.VMEM((_LANES,), jnp.int32),
            pltpu.VMEM((_LANES,), jnp.int32),
            pltpu.VMEM((_LANES,), jnp.int32),
            pltpu.VMEM((_LANES, n), jnp.float32),
            pltpu.VMEM((_LANES, n), jnp.float32),
            pltpu.VMEM((_LANES, n), jnp.float32),
            pltpu.SemaphoreType.DMA,
            pltpu.SemaphoreType.DMA,
            pltpu.SemaphoreType.DMA,
        ],
    )
    a0, a1, a2 = sc_gather(is_tab, it_tab, im_tab, channel_embed, pos_tab, mtab)
    a_4d = jnp.concatenate([a0, a1, a2], axis=1).reshape(b, t, s, 3 * n)

    s_tab = pl.pallas_call(
        functools.partial(_spatial_kernel, w_cnt=w),
        in_specs=[pl.BlockSpec(memory_space=pltpu.SMEM)],
        out_specs=pl.BlockSpec((hw, n), lambda: (0, 0)),
        out_shape=jax.ShapeDtypeStruct((hw, n), jnp.float32),
    )(gsd)
    s_3d = s_tab.reshape(h, w, n)

    h_blk = 4
    nhb = h // h_blk
    out = pl.pallas_call(
        _stream_kernel,
        grid=(b * nhb,),
        in_specs=[
            pl.BlockSpec((1, h_blk, w, t, s, d),
                         lambda i: (i // nhb, i % nhb, 0, 0, 0, 0)),
            pl.BlockSpec((1, t, s, 3 * n), lambda i: (i // nhb, 0, 0, 0)),
            pl.BlockSpec((h_blk, w, n), lambda i: (i % nhb, 0, 0)),
        ],
        out_specs=pl.BlockSpec((1, h_blk, w, t, s, d),
                               lambda i: (i // nhb, i % nhb, 0, 0, 0, 0)),
        out_shape=jax.ShapeDtypeStruct(tokens.shape, jnp.float32),
        compiler_params=pltpu.CompilerParams(
            dimension_semantics=("parallel",)),
    )(tokens, a_4d, s_3d)
    return out
